# Initial kernel scaffold; baseline (speedup 1.0000x reference)
import jax
import jax.numpy as jnp
from jax.experimental import pallas as pl


def kernel(inputs, labels):
    raise NotImplementedError("write your pallas kernel here")


# speedup vs baseline: 3.5927x; 3.5927x over previous
import functools

import jax
import jax.numpy as jnp
from jax.experimental import pallas as pl

_N = 16384
_C = 1000
_B = 2048
_G = _N // _B


def _body(lab_ref, x_ref, sums_ref, cnt_ref):
    x = x_ref[...]
    lab = lab_ref[...]
    col = jax.lax.broadcasted_iota(jnp.int32, (_B, _C), 1)
    onehot = col == lab
    sumsq1 = jnp.sum(x * x, axis=1, keepdims=True) + 1.0
    a = jnp.sum(jnp.where(onehot, sumsq1 - 2.0 * x, 0.0), axis=0, keepdims=True)
    cnt = jnp.sum(jnp.where(onehot, 1.0, 0.0), axis=0, keepdims=True)

    @pl.when(pl.program_id(0) == 0)
    def _():
        sums_ref[...] = jnp.zeros_like(sums_ref)
        cnt_ref[...] = jnp.zeros_like(cnt_ref)

    sums_ref[...] += a * (1.0 / _C)
    cnt_ref[...] += cnt


@functools.partial(jax.jit, static_argnames=("interpret",))
def kernel(inputs, labels, interpret=False):
    labels2d = labels.astype(jnp.int32).reshape(_N, 1)
    sums, cnt = pl.pallas_call(
        _body,
        grid=(_G,),
        in_specs=[
            pl.BlockSpec((_B, 1), lambda i: (i, 0)),
            pl.BlockSpec((_B, _C), lambda i: (i, 0)),
        ],
        out_specs=[
            pl.BlockSpec((1, _C), lambda i: (0, 0)),
            pl.BlockSpec((1, _C), lambda i: (0, 0)),
        ],
        out_shape=[
            jax.ShapeDtypeStruct((1, _C), jnp.float32),
            jax.ShapeDtypeStruct((1, _C), jnp.float32),
        ],
        interpret=interpret,
    )(labels2d, inputs)
    return (sums.reshape(_C), cnt.reshape(_C))
